# initial kernel scaffold (unmeasured)
import jax
import jax.numpy as jnp
from jax import lax
from jax.experimental import pallas as pl
from jax.experimental.pallas import tpu as pltpu

N_DEV = 32


def kernel(x, Wq, Wo, K_ext, V_ext):
    B, Sq, D = x.shape
    _, Skv, Hq, Dh = K_ext.shape
    BH = B * Hq
    bf16 = jnp.bfloat16

    xb = x.astype(bf16)
    Wqb = Wq.astype(bf16)
    Wob = Wo.astype(bf16)
    K2 = jnp.transpose(K_ext, (0, 2, 1, 3)).reshape(BH, Skv, Dh).astype(bf16)
    V2 = jnp.transpose(V_ext, (0, 2, 1, 3)).reshape(BH, Skv, Dh).astype(bf16)

    def body(x_ref, wq_ref, wo_ref, k_ref, v_ref, out_ref,
             kv, q_ref, acc, m_ref, l_ref, send_sems, recv_sems):
        my = lax.axis_index("i")
        right = lax.rem(my + 1, N_DEV)

        for b in range(B):
            Qb = lax.dot_general(
                x_ref[b], wq_ref[...], (((1,), (0,)), ((), ())),
                preferred_element_type=jnp.float32,
            ).astype(bf16)
            for hh in range(Hq):
                q_ref[b * Hq + hh] = Qb[:, hh * Dh:(hh + 1) * Dh]

        m_ref[...] = jnp.full(m_ref.shape, -1e30, jnp.float32)
        l_ref[...] = jnp.zeros(l_ref.shape, jnp.float32)
        acc[...] = jnp.zeros(acc.shape, jnp.float32)

        kv[0, 0:BH] = k_ref[...]
        kv[0, BH:2 * BH] = v_ref[...]

        def absorb_chunk(slot):
            def tbody(t, carry):
                Qt = q_ref[t]
                Kt = kv[slot, t]
                Vt = kv[slot, BH + t]
                S = lax.dot_general(
                    Qt, Kt, (((1,), (1,)), ((), ())),
                    preferred_element_type=jnp.float32,
                ) * 0.125
                m_prev = m_ref[t]
                m_new = jnp.maximum(m_prev, jnp.max(S, axis=1, keepdims=True))
                alpha = jnp.exp(m_prev - m_new)
                P = jnp.exp(S - m_new)
                l_ref[t] = l_ref[t] * alpha + jnp.sum(P, axis=1, keepdims=True)
                pv = lax.dot_general(
                    P.astype(bf16), Vt, (((1,), (0,)), ((), ())),
                    preferred_element_type=jnp.float32,
                )
                acc[t] = acc[t] * alpha + pv
                m_ref[t] = m_new
                return carry
            lax.fori_loop(0, BH, tbody, 0)

        rdmas = []
        for h in range(N_DEV):
            if h >= 1:
                rdmas[h - 1].wait()
            if h < N_DEV - 1:
                r = pltpu.make_async_remote_copy(
                    src_ref=kv.at[h],
                    dst_ref=kv.at[h + 1],
                    send_sem=send_sems.at[h],
                    recv_sem=recv_sems.at[h],
                    device_id=(right,),
                    device_id_type=pl.DeviceIdType.MESH,
                )
                r.start()
                rdmas.append(r)
            absorb_chunk(h)

        for b in range(B):
            ob = jnp.zeros((Sq, D), jnp.float32)
            for hh in range(Hq):
                t = b * Hq + hh
                o_bh = (acc[t] / l_ref[t]).astype(bf16)
                ob = ob + lax.dot_general(
                    o_bh, wo_ref[hh * Dh:(hh + 1) * Dh, :],
                    (((1,), (0,)), ((), ())),
                    preferred_element_type=jnp.float32,
                )
            out_ref[b] = ob

    return pl.pallas_call(
        body,
        out_shape=jax.ShapeDtypeStruct((B, Sq, D), jnp.float32),
        in_specs=[pl.BlockSpec(memory_space=pltpu.VMEM)] * 5,
        out_specs=pl.BlockSpec(memory_space=pltpu.VMEM),
        scratch_shapes=[
            pltpu.VMEM((N_DEV, 2 * BH, Skv, Dh), bf16),
            pltpu.VMEM((BH, Sq, Dh), bf16),
            pltpu.VMEM((BH, Sq, Dh), jnp.float32),
            pltpu.VMEM((BH, Sq, 1), jnp.float32),
            pltpu.VMEM((BH, Sq, 1), jnp.float32),
            pltpu.SemaphoreType.DMA((N_DEV - 1,)),
            pltpu.SemaphoreType.DMA((N_DEV - 1,)),
        ],
        compiler_params=pltpu.CompilerParams(collective_id=0),
    )(xb, Wqb, Wob, K2, V2)


# baseline (device time: 435948 ns/iter reference)
import jax
import jax.numpy as jnp
from jax import lax
from jax.experimental import pallas as pl
from jax.experimental.pallas import tpu as pltpu

N_DEV = 32


def kernel(x, Wq, Wo, K_ext, V_ext):
    B, Sq, D = x.shape
    _, Skv, Hq, Dh = K_ext.shape
    BH = B * Hq
    bf16 = jnp.bfloat16

    xb = x.astype(bf16)
    Wqb = Wq.astype(bf16)
    Wob = Wo.astype(bf16)
    K2 = jnp.transpose(K_ext, (0, 2, 1, 3)).reshape(BH, Skv, Dh).astype(bf16)
    V2 = jnp.transpose(V_ext, (0, 2, 1, 3)).reshape(BH, Skv, Dh).astype(bf16)

    def body(x_ref, wq_ref, wo_ref, k_ref, v_ref, out_ref,
             kv, q_ref, acc, m_ref, l_ref, send_sems, recv_sems):
        my = lax.axis_index("i")
        right = lax.rem(my + 1, N_DEV)

        for b in range(B):
            Qb = lax.dot_general(
                x_ref[b], wq_ref[...], (((1,), (0,)), ((), ())),
                preferred_element_type=jnp.float32,
            ).astype(bf16)
            for hh in range(Hq):
                q_ref[b * Hq + hh] = Qb[:, hh * Dh:(hh + 1) * Dh]

        m_ref[...] = jnp.full(m_ref.shape, -1e30, jnp.float32)
        l_ref[...] = jnp.zeros(l_ref.shape, jnp.float32)
        acc[...] = jnp.zeros(acc.shape, jnp.float32)

        kv[0, 0:BH] = k_ref[...]
        kv[0, BH:2 * BH] = v_ref[...]

        def absorb_chunk(slot):
            def tbody(t, carry):
                Qt = q_ref[t]
                Kt = kv[slot, t]
                Vt = kv[slot, BH + t]
                S = lax.dot_general(
                    Qt, Kt, (((1,), (1,)), ((), ())),
                    preferred_element_type=jnp.float32,
                ) * 0.125
                m_prev = m_ref[t]
                m_new = jnp.maximum(m_prev, jnp.max(S, axis=1, keepdims=True))
                alpha = jnp.exp(m_prev - m_new)
                P = jnp.exp(S - m_new)
                l_ref[t] = l_ref[t] * alpha + jnp.sum(P, axis=1, keepdims=True)
                pv = lax.dot_general(
                    P.astype(bf16), Vt, (((1,), (0,)), ((), ())),
                    preferred_element_type=jnp.float32,
                )
                acc[t] = acc[t] * alpha + pv
                m_ref[t] = m_new
                return carry
            lax.fori_loop(0, BH, tbody, 0)

        rdmas = []
        for h in range(N_DEV):
            if h >= 1:
                rdmas[h - 1].wait()
            if h < N_DEV - 1:
                r = pltpu.make_async_remote_copy(
                    src_ref=kv.at[h],
                    dst_ref=kv.at[h + 1],
                    send_sem=send_sems.at[h],
                    recv_sem=recv_sems.at[h],
                    device_id=(right,),
                    device_id_type=pl.DeviceIdType.MESH,
                )
                r.start()
                rdmas.append(r)
            absorb_chunk(h)

        for b in range(B):
            ob = jnp.zeros((Sq, D), jnp.float32)
            for hh in range(Hq):
                t = b * Hq + hh
                o_bh = (acc[t] / l_ref[t]).astype(bf16)
                ob = ob + lax.dot_general(
                    o_bh, wo_ref[hh * Dh:(hh + 1) * Dh, :],
                    (((1,), (0,)), ((), ())),
                    preferred_element_type=jnp.float32,
                )
            out_ref[b] = ob

    return pl.pallas_call(
        body,
        out_shape=jax.ShapeDtypeStruct((B, Sq, D), jnp.float32),
        in_specs=[pl.BlockSpec(memory_space=pltpu.VMEM)] * 5,
        out_specs=pl.BlockSpec(memory_space=pltpu.VMEM),
        scratch_shapes=[
            pltpu.VMEM((N_DEV, 2 * BH, Skv, Dh), bf16),
            pltpu.VMEM((BH, Sq, Dh), bf16),
            pltpu.VMEM((BH, Sq, Dh), jnp.float32),
            pltpu.VMEM((BH, Sq, 1), jnp.float32),
            pltpu.VMEM((BH, Sq, 1), jnp.float32),
            pltpu.SemaphoreType.DMA((N_DEV - 1,)),
            pltpu.SemaphoreType.DMA((N_DEV - 1,)),
        ],
        compiler_params=pltpu.CompilerParams(
            vmem_limit_bytes=64 * 1024 * 1024,
        ),
    )(xb, Wqb, Wob, K2, V2)


# device time: 421346 ns/iter; 1.0347x vs baseline; 1.0347x over previous
import jax
import jax.numpy as jnp
from jax import lax
from jax.experimental import pallas as pl
from jax.experimental.pallas import tpu as pltpu

N_DEV = 32


def kernel(x, Wq, Wo, K_ext, V_ext):
    B, Sq, D = x.shape
    _, Skv, Hq, Dh = K_ext.shape
    BH = B * Hq
    bf16 = jnp.bfloat16

    xb = x.astype(bf16)
    Wqb = Wq.astype(bf16)
    Wob = Wo.astype(bf16)
    K2 = jnp.transpose(K_ext, (0, 2, 1, 3)).reshape(BH, Skv, Dh).astype(bf16)
    V2 = jnp.transpose(V_ext, (0, 2, 1, 3)).reshape(BH, Skv, Dh).astype(bf16)

    def body(x_ref, wq_ref, wo_ref, k_ref, v_ref, out_ref,
             kvR, kvL, q_ref, acc, m_ref, l_ref,
             send_semsR, recv_semsR, send_semsL, recv_semsL):
        my = lax.axis_index("i")
        right = lax.rem(my + 1, N_DEV)
        left = lax.rem(my + N_DEV - 1, N_DEV)

        for b in range(B):
            Qb = lax.dot_general(
                x_ref[b], wq_ref[...], (((1,), (0,)), ((), ())),
                preferred_element_type=jnp.float32,
            ).astype(bf16)
            for hh in range(Hq):
                q_ref[b * Hq + hh] = Qb[:, hh * Dh:(hh + 1) * Dh]

        m_ref[...] = jnp.full(m_ref.shape, -1e30, jnp.float32)
        l_ref[...] = jnp.zeros(l_ref.shape, jnp.float32)
        acc[...] = jnp.zeros(acc.shape, jnp.float32)

        kvR[0, 0:BH] = k_ref[...]
        kvR[0, BH:2 * BH] = v_ref[...]
        kvL[0, 0:BH] = k_ref[...]
        kvL[0, BH:2 * BH] = v_ref[...]

        def absorb_chunk(kv, slot):
            def tbody(t, carry):
                Qt = q_ref[t]
                Kt = kv[slot, t]
                Vt = kv[slot, BH + t]
                S = lax.dot_general(
                    Qt, Kt, (((1,), (1,)), ((), ())),
                    preferred_element_type=jnp.float32,
                ) * 0.125
                m_prev = m_ref[t]
                m_new = jnp.maximum(m_prev, jnp.max(S, axis=1, keepdims=True))
                alpha = jnp.exp(m_prev - m_new)
                P = jnp.exp(S - m_new)
                l_ref[t] = l_ref[t] * alpha + jnp.sum(P, axis=1, keepdims=True)
                pv = lax.dot_general(
                    P.astype(bf16), Vt, (((1,), (0,)), ((), ())),
                    preferred_element_type=jnp.float32,
                )
                acc[t] = acc[t] * alpha + pv
                m_ref[t] = m_new
                return carry
            lax.fori_loop(0, BH, tbody, 0)

        HR = (N_DEV - 1) // 2
        HL = N_DEV - 1 - HR
        rdmasR, rdmasL = [], []
        for h in range(HL + 1):
            if 1 <= h <= HR:
                rdmasR[h - 1].wait()
            if 1 <= h <= HL:
                rdmasL[h - 1].wait()
            if h < HR:
                r = pltpu.make_async_remote_copy(
                    src_ref=kvR.at[h],
                    dst_ref=kvR.at[h + 1],
                    send_sem=send_semsR.at[h],
                    recv_sem=recv_semsR.at[h],
                    device_id=(right,),
                    device_id_type=pl.DeviceIdType.MESH,
                )
                r.start()
                rdmasR.append(r)
            if h < HL:
                r = pltpu.make_async_remote_copy(
                    src_ref=kvL.at[h],
                    dst_ref=kvL.at[h + 1],
                    send_sem=send_semsL.at[h],
                    recv_sem=recv_semsL.at[h],
                    device_id=(left,),
                    device_id_type=pl.DeviceIdType.MESH,
                )
                r.start()
                rdmasL.append(r)
            if h == 0:
                absorb_chunk(kvR, 0)
            else:
                if h <= HR:
                    absorb_chunk(kvR, h)
                if h <= HL:
                    absorb_chunk(kvL, h)

        for b in range(B):
            ob = jnp.zeros((Sq, D), jnp.float32)
            for hh in range(Hq):
                t = b * Hq + hh
                o_bh = (acc[t] / l_ref[t]).astype(bf16)
                ob = ob + lax.dot_general(
                    o_bh, wo_ref[hh * Dh:(hh + 1) * Dh, :],
                    (((1,), (0,)), ((), ())),
                    preferred_element_type=jnp.float32,
                )
            out_ref[b] = ob

    return pl.pallas_call(
        body,
        out_shape=jax.ShapeDtypeStruct((B, Sq, D), jnp.float32),
        in_specs=[pl.BlockSpec(memory_space=pltpu.VMEM)] * 5,
        out_specs=pl.BlockSpec(memory_space=pltpu.VMEM),
        scratch_shapes=[
            pltpu.VMEM((16, 2 * BH, Skv, Dh), bf16),
            pltpu.VMEM((17, 2 * BH, Skv, Dh), bf16),
            pltpu.VMEM((BH, Sq, Dh), bf16),
            pltpu.VMEM((BH, Sq, Dh), jnp.float32),
            pltpu.VMEM((BH, Sq, 1), jnp.float32),
            pltpu.VMEM((BH, Sq, 1), jnp.float32),
            pltpu.SemaphoreType.DMA((15,)),
            pltpu.SemaphoreType.DMA((15,)),
            pltpu.SemaphoreType.DMA((16,)),
            pltpu.SemaphoreType.DMA((16,)),
        ],
        compiler_params=pltpu.CompilerParams(
            vmem_limit_bytes=64 * 1024 * 1024,
        ),
    )(xb, Wqb, Wob, K2, V2)


# device time: 412914 ns/iter; 1.0558x vs baseline; 1.0204x over previous
import jax
import jax.numpy as jnp
from jax import lax
from jax.experimental import pallas as pl
from jax.experimental.pallas import tpu as pltpu

N_DEV = 32


def kernel(x, Wq, Wo, K_ext, V_ext):
    B, Sq, D = x.shape
    _, Skv, Hq, Dh = K_ext.shape
    BH = B * Hq
    bf16 = jnp.bfloat16

    xb = x.astype(bf16)
    Wqb = Wq.astype(bf16)
    Wob = Wo.astype(bf16)
    K2 = jnp.transpose(K_ext, (0, 2, 1, 3)).reshape(BH, Skv, Dh).astype(bf16)
    V2 = jnp.transpose(V_ext, (0, 2, 1, 3)).reshape(BH, Skv, Dh).astype(bf16)

    def body(x_ref, wq_ref, wo_ref, k_ref, v_ref, out_ref,
             kvR, kvL, q_ref, acc, m_ref, l_ref,
             send_semsR, recv_semsR, send_semsL, recv_semsL):
        my = lax.axis_index("i")
        right = lax.rem(my + 1, N_DEV)
        left = lax.rem(my + N_DEV - 1, N_DEV)

        for b in range(B):
            Qb = lax.dot_general(
                x_ref[b], wq_ref[...], (((1,), (0,)), ((), ())),
                preferred_element_type=jnp.float32,
            ).astype(bf16)
            for hh in range(Hq):
                q_ref[b * Hq + hh] = Qb[:, hh * Dh:(hh + 1) * Dh]

        m_ref[...] = jnp.full(m_ref.shape, -1e30, jnp.float32)
        l_ref[...] = jnp.zeros(l_ref.shape, jnp.float32)
        acc[...] = jnp.zeros(acc.shape, jnp.float32)

        kvR[0, 0:BH] = k_ref[...]
        kvR[0, BH:2 * BH] = v_ref[...]
        kvL[0, 0:BH] = k_ref[...]
        kvL[0, BH:2 * BH] = v_ref[...]

        def absorb_chunk(kv, slot):
            for t in range(BH):
                Qt = q_ref[t]
                Kt = kv[slot, t]
                Vt = kv[slot, BH + t]
                S = lax.dot_general(
                    Qt, Kt, (((1,), (1,)), ((), ())),
                    preferred_element_type=jnp.float32,
                ) * 0.125
                m_prev = m_ref[t]
                m_new = jnp.maximum(m_prev, jnp.max(S, axis=1, keepdims=True))
                alpha = jnp.exp(m_prev - m_new)
                P = jnp.exp(S - m_new)
                l_ref[t] = l_ref[t] * alpha + jnp.sum(P, axis=1, keepdims=True)
                pv = lax.dot_general(
                    P.astype(bf16), Vt, (((1,), (0,)), ((), ())),
                    preferred_element_type=jnp.float32,
                )
                acc[t] = acc[t] * alpha + pv
                m_ref[t] = m_new

        HR = (N_DEV - 1) // 2
        HL = N_DEV - 1 - HR
        rdmasR, rdmasL = [], []
        for h in range(HL + 1):
            if 1 <= h <= HR:
                rdmasR[h - 1].wait()
            if 1 <= h <= HL:
                rdmasL[h - 1].wait()
            if h < HR:
                r = pltpu.make_async_remote_copy(
                    src_ref=kvR.at[h],
                    dst_ref=kvR.at[h + 1],
                    send_sem=send_semsR.at[h],
                    recv_sem=recv_semsR.at[h],
                    device_id=(right,),
                    device_id_type=pl.DeviceIdType.MESH,
                )
                r.start()
                rdmasR.append(r)
            if h < HL:
                r = pltpu.make_async_remote_copy(
                    src_ref=kvL.at[h],
                    dst_ref=kvL.at[h + 1],
                    send_sem=send_semsL.at[h],
                    recv_sem=recv_semsL.at[h],
                    device_id=(left,),
                    device_id_type=pl.DeviceIdType.MESH,
                )
                r.start()
                rdmasL.append(r)
            if h == 0:
                absorb_chunk(kvR, 0)
            else:
                if h <= HR:
                    absorb_chunk(kvR, h)
                if h <= HL:
                    absorb_chunk(kvL, h)

        for b in range(B):
            ob = jnp.zeros((Sq, D), jnp.float32)
            for hh in range(Hq):
                t = b * Hq + hh
                o_bh = (acc[t] / l_ref[t]).astype(bf16)
                ob = ob + lax.dot_general(
                    o_bh, wo_ref[hh * Dh:(hh + 1) * Dh, :],
                    (((1,), (0,)), ((), ())),
                    preferred_element_type=jnp.float32,
                )
            out_ref[b] = ob

    return pl.pallas_call(
        body,
        out_shape=jax.ShapeDtypeStruct((B, Sq, D), jnp.float32),
        in_specs=[pl.BlockSpec(memory_space=pltpu.VMEM)] * 5,
        out_specs=pl.BlockSpec(memory_space=pltpu.VMEM),
        scratch_shapes=[
            pltpu.VMEM((16, 2 * BH, Skv, Dh), bf16),
            pltpu.VMEM((17, 2 * BH, Skv, Dh), bf16),
            pltpu.VMEM((BH, Sq, Dh), bf16),
            pltpu.VMEM((BH, Sq, Dh), jnp.float32),
            pltpu.VMEM((BH, Sq, 1), jnp.float32),
            pltpu.VMEM((BH, Sq, 1), jnp.float32),
            pltpu.SemaphoreType.DMA((15,)),
            pltpu.SemaphoreType.DMA((15,)),
            pltpu.SemaphoreType.DMA((16,)),
            pltpu.SemaphoreType.DMA((16,)),
        ],
        compiler_params=pltpu.CompilerParams(
            vmem_limit_bytes=64 * 1024 * 1024,
        ),
    )(xb, Wqb, Wob, K2, V2)


# device time: 230153 ns/iter; 1.8942x vs baseline; 1.7941x over previous
import jax
import jax.numpy as jnp
from jax import lax
from jax.experimental import pallas as pl
from jax.experimental.pallas import tpu as pltpu

N_DEV = 32


def kernel(x, Wq, Wo, K_ext, V_ext):
    B, Sq, D = x.shape
    _, Skv, Hq, Dh = K_ext.shape
    BH = B * Hq
    bf16 = jnp.bfloat16

    xb = x.astype(bf16)
    Wqb = Wq.astype(bf16)
    Wob = Wo.astype(bf16)
    K2 = K_ext.reshape(B, Skv, Hq * Dh).astype(bf16)
    V2 = V_ext.reshape(B, Skv, Hq * Dh).astype(bf16)

    def body(x_ref, wq_ref, wo_ref, k_ref, v_ref, out_ref,
             kv, q_ref, acc, m_ref, l_ref,
             send_semsK, recv_semsK, send_semsV, recv_semsV):
        my = lax.axis_index("i")
        right = lax.rem(my + 1, N_DEV)

        for b in range(B):
            q_ref[b] = lax.dot_general(
                x_ref[b], wq_ref[...], (((1,), (0,)), ((), ())),
                preferred_element_type=jnp.float32,
            ).astype(bf16)

        m_ref[...] = jnp.full(m_ref.shape, -1e30, jnp.float32)
        l_ref[...] = jnp.zeros(l_ref.shape, jnp.float32)
        acc[...] = jnp.zeros(acc.shape, jnp.float32)

        kv[0, 0:B] = k_ref[...]
        kv[0, B:2 * B] = v_ref[...]

        def absorb_chunk(slot):
            for b in range(B):
                for hh in range(Hq):
                    t = b * Hq + hh
                    lo, hi = hh * Dh, (hh + 1) * Dh
                    Qt = q_ref[b, :, lo:hi]
                    Kt = kv[slot, b, :, lo:hi]
                    Vt = kv[slot, B + b, :, lo:hi]
                    S = lax.dot_general(
                        Qt, Kt, (((1,), (1,)), ((), ())),
                        preferred_element_type=jnp.float32,
                    ) * 0.125
                    m_prev = m_ref[t]
                    m_new = jnp.maximum(
                        m_prev, jnp.max(S, axis=1, keepdims=True))
                    alpha = jnp.exp(m_prev - m_new)
                    P = jnp.exp(S - m_new)
                    l_ref[t] = (l_ref[t] * alpha
                                + jnp.sum(P, axis=1, keepdims=True))
                    pv = lax.dot_general(
                        P.astype(bf16), Vt, (((1,), (0,)), ((), ())),
                        preferred_element_type=jnp.float32,
                    )
                    acc[t] = acc[t] * alpha + pv
                    m_ref[t] = m_new

        def half_copy(h, lo, hi, send_sems, recv_sems):
            return pltpu.make_async_remote_copy(
                src_ref=kv.at[h, lo:hi],
                dst_ref=kv.at[h + 1, lo:hi],
                send_sem=send_sems.at[h],
                recv_sem=recv_sems.at[h],
                device_id=(right,),
                device_id_type=pl.DeviceIdType.MESH,
            )

        rK, rV = [], []
        for h in range(N_DEV):
            if h >= 1:
                rK[h - 1].wait()
            if h < N_DEV - 1:
                r = half_copy(h, 0, B, send_semsK, recv_semsK)
                r.start()
                rK.append(r)
            if h >= 1:
                rV[h - 1].wait()
            if h < N_DEV - 1:
                r = half_copy(h, B, 2 * B, send_semsV, recv_semsV)
                r.start()
                rV.append(r)
            absorb_chunk(h)

        for b in range(B):
            ob = jnp.zeros((Sq, D), jnp.float32)
            for hh in range(Hq):
                t = b * Hq + hh
                o_bh = (acc[t] / l_ref[t]).astype(bf16)
                ob = ob + lax.dot_general(
                    o_bh, wo_ref[hh * Dh:(hh + 1) * Dh, :],
                    (((1,), (0,)), ((), ())),
                    preferred_element_type=jnp.float32,
                )
            out_ref[b] = ob

    return pl.pallas_call(
        body,
        out_shape=jax.ShapeDtypeStruct((B, Sq, D), jnp.float32),
        in_specs=[pl.BlockSpec(memory_space=pltpu.VMEM)] * 5,
        out_specs=pl.BlockSpec(memory_space=pltpu.VMEM),
        scratch_shapes=[
            pltpu.VMEM((N_DEV, 2 * B, Skv, Hq * Dh), bf16),
            pltpu.VMEM((B, Sq, Hq * Dh), bf16),
            pltpu.VMEM((BH, Sq, Dh), jnp.float32),
            pltpu.VMEM((BH, Sq, 1), jnp.float32),
            pltpu.VMEM((BH, Sq, 1), jnp.float32),
            pltpu.SemaphoreType.DMA((N_DEV - 1,)),
            pltpu.SemaphoreType.DMA((N_DEV - 1,)),
            pltpu.SemaphoreType.DMA((N_DEV - 1,)),
            pltpu.SemaphoreType.DMA((N_DEV - 1,)),
        ],
        compiler_params=pltpu.CompilerParams(
            vmem_limit_bytes=64 * 1024 * 1024,
        ),
    )(xb, Wqb, Wob, K2, V2)
